# baseline (speedup 1.0000x reference)
import jax
import jax.numpy as jnp
from jax.experimental import pallas as pl
from jax.experimental.pallas import tpu as pltpu

_N_ROWS = 65536
_N_COLS = 512
_ONES_ROWS = 512
_BLOCK_ROWS = 1024


def _fill_kernel(o_ref):
    i = pl.program_id(0)
    row = jax.lax.broadcasted_iota(jnp.int32, o_ref.shape, 0) + i * _BLOCK_ROWS
    o_ref[...] = (row < _ONES_ROWS).astype(jnp.float32)


def kernel(x):
    return pl.pallas_call(
        _fill_kernel,
        grid=(_N_ROWS // _BLOCK_ROWS,),
        out_specs=pl.BlockSpec((_BLOCK_ROWS, _N_COLS), lambda i: (i, 0)),
        out_shape=jax.ShapeDtypeStruct((_N_ROWS, _N_COLS), x.dtype),
        compiler_params=pltpu.CompilerParams(
            dimension_semantics=("parallel",),
        ),
    )()


# speedup vs baseline: 1.1215x; 1.1215x over previous
import jax
import jax.numpy as jnp
from jax.experimental import pallas as pl
from jax.experimental.pallas import tpu as pltpu

_N_ROWS = 65536
_N_COLS = 512
_ONES_ROWS = 512
_COPY_ROWS = 2048
_TPL_ROWS = _ONES_ROWS + _COPY_ROWS


def _dma_fill_kernel(o_ref, tpl, sem):
    row = jax.lax.broadcasted_iota(jnp.int32, tpl.shape, 0)
    tpl[...] = (row < _ONES_ROWS).astype(jnp.float32)
    copies = []
    c = pltpu.make_async_copy(
        tpl.at[pl.ds(0, _COPY_ROWS), :], o_ref.at[pl.ds(0, _COPY_ROWS), :], sem
    )
    c.start()
    copies.append(c)
    for i in range(1, _N_ROWS // _COPY_ROWS):
        c = pltpu.make_async_copy(
            tpl.at[pl.ds(_ONES_ROWS, _COPY_ROWS), :],
            o_ref.at[pl.ds(i * _COPY_ROWS, _COPY_ROWS), :],
            sem,
        )
        c.start()
        copies.append(c)
    for c in copies:
        c.wait()


def kernel(x):
    return pl.pallas_call(
        _dma_fill_kernel,
        out_specs=pl.BlockSpec(memory_space=pl.ANY),
        out_shape=jax.ShapeDtypeStruct((_N_ROWS, _N_COLS), x.dtype),
        scratch_shapes=[
            pltpu.VMEM((_TPL_ROWS, _N_COLS), jnp.float32),
            pltpu.SemaphoreType.DMA,
        ],
    )()
